# baseline (speedup 1.0000x reference)
import functools

import jax
import jax.numpy as jnp
from jax import lax
from jax.experimental import pallas as pl
from jax.experimental.pallas import tpu as pltpu
from jax.experimental.pallas import tpu_sc as plsc

NC = 2
NS = 16
L = 16
NW = NC * NS

C = 128
W = 128


def kernel(x, table, scale):
    B, S = x.shape
    V, D = table.shape
    N = B * S

    xi = x.reshape(N).astype(jnp.int32)
    t2 = table.reshape(V // 2, 2 * D)
    scale_vec = jnp.broadcast_to(scale.astype(jnp.float32), (L,))

    t_per_w = N // NW
    n_chunks = t_per_w // C
    n_g = n_chunks // 2

    mesh = plsc.VectorSubcoreMesh(core_axis_name="c", subcore_axis_name="s")

    @functools.partial(
        pl.kernel,
        out_type=jax.ShapeDtypeStruct((N, D), jnp.float32),
        mesh=mesh,
        scratch_types=[
            pltpu.VMEM((C,), jnp.int32),
            pltpu.VMEM((C,), jnp.int32),
            pltpu.VMEM((C + L,), jnp.int32),
            pltpu.VMEM((C + L,), jnp.int32),
            pltpu.VMEM((C, 2 * D), jnp.float32),
            pltpu.VMEM((C, 2 * D), jnp.float32),
            pltpu.VMEM((C, D), jnp.float32),
            pltpu.VMEM((C, D), jnp.float32),
            pltpu.VMEM((L,), jnp.float32),
            pltpu.SemaphoreType.DMA,
            pltpu.SemaphoreType.DMA,
            pltpu.SemaphoreType.DMA,
            pltpu.SemaphoreType.DMA,
        ],
        compiler_params=pltpu.CompilerParams(use_tc_tiling_on_sc=True),
    )
    def emb_kernel(idx_hbm, t2_hbm, scale_hbm, out_hbm,
                   idx2_0, idx2_1, par_0, par_1, rows_0, rows_1,
                   outv_0, outv_1, scale_v, gsem0, gsem1, osem0, osem1):
        wid = lax.axis_index("s") * NC + lax.axis_index("c")
        base = wid * t_per_w
        pltpu.sync_copy(scale_hbm, scale_v)
        sv = scale_v[...]

        bufs = ((idx2_0, par_0, rows_0, outv_0, gsem0, osem0),
                (idx2_1, par_1, rows_1, outv_1, gsem1, osem1))

        def fire_gather(ci, idx2, par, rows, gsem):
            pltpu.sync_copy(idx_hbm.at[pl.ds(base + ci * C, C)], idx2)
            for t in range(C // L):
                sl = pl.ds(t * L, L)
                v = idx2[sl]
                par[sl] = (v & 1) << 6
                idx2[sl] = v >> 1
            for w in range(C // W):
                pltpu.async_copy(
                    t2_hbm.at[idx2.at[pl.ds(w * W, W)]],
                    rows.at[pl.ds(w * W, W)], gsem)

        def wait_gather(idx2, rows, gsem):
            for w in range(C // W):
                pltpu.make_async_copy(
                    t2_hbm.at[idx2.at[pl.ds(w * W, W)]],
                    rows.at[pl.ds(w * W, W)], gsem).wait()

        for b in range(2):
            idx2, par, rows, _, gsem, _ = bufs[b]
            fire_gather(b, idx2, par, rows, gsem)

        @pl.loop(0, n_g)
        def _(g):
            for b in range(2):
                idx2, par, rows, outv, gsem, osem = bufs[b]
                ci = g * 2 + b
                wait_gather(idx2, rows, gsem)

                @pl.when(g >= 1)
                def _():
                    pltpu.make_async_copy(
                        outv, out_hbm.at[pl.ds(base + (ci - 2) * C, C)],
                        osem).wait()

                @pl.loop(0, C)
                def _(r):
                    c0 = par[pl.ds(r, L)][0]
                    for jj in range(D // L):
                        outv[r, pl.ds(jj * L, L)] = (
                            rows[r, pl.ds(c0 + jj * L, L)] * sv)

                pltpu.async_copy(
                    outv, out_hbm.at[pl.ds(base + ci * C, C)], osem)

                @pl.when(g < n_g - 1)
                def _():
                    fire_gather(ci + 2, idx2, par, rows, gsem)

        for b in range(2):
            _, _, _, outv, _, osem = bufs[b]
            pltpu.make_async_copy(
                outv, out_hbm.at[pl.ds(base + (n_chunks - 2 + b) * C, C)],
                osem).wait()

    return emb_kernel(xi, t2, scale_vec).reshape(B, S, D)
